# baseline (speedup 1.0000x reference)
import jax
import jax.numpy as jnp
from jax import lax
from jax.experimental import pallas as pl
from jax.experimental.pallas import tpu as pltpu
from jax.experimental.pallas import tpu_sc as plsc

B, S, H = 8, 4096, 1024
CLASS_ID, TEXT_ID = 1, 2
C = 16 + B - 1
CROWS = 24
CPAD = 32
SCW = 40
LANES = 16
MAX_TEXT = S - (16 * 8 + 2)
CS = 256
NCHUNK = S // CS


def _idx_body(ids_ref, attn_ref, scal_ref):
    ids = ids_ref[...]
    attn = attn_ref[...]
    pos = lax.broadcasted_iota(jnp.int32, (B, S), 1)
    cmask = ids == CLASS_ID
    ncl = jnp.sum(jnp.where(cmask, 1, 0), axis=1, keepdims=True)
    ts = jnp.min(jnp.where(ids == TEXT_ID, pos, S), axis=1, keepdims=True)
    ts = jnp.where(ts >= S, 0, ts)
    eos = jnp.max(jnp.where(attn != 0, pos, -1), axis=1, keepdims=True)
    eos = jnp.where(eos < 0, S - 1, eos)
    prev = jnp.full((B, 1), -1, jnp.int32)
    for c in range(CROWS):
        cur = jnp.min(jnp.where(cmask & (pos > prev), pos, S), axis=1,
                      keepdims=True)
        scal_ref[:, c:c + 1] = jnp.where(cur < S, cur, 0)
        prev = cur
    for c in range(CROWS, 32):
        scal_ref[:, c:c + 1] = jnp.zeros((B, 1), jnp.int32)
    scal_ref[:, 32:33] = ts
    scal_ref[:, 33:34] = eos
    scal_ref[:, 34:35] = ncl
    for c in range(35, SCW):
        scal_ref[:, c:c + 1] = jnp.zeros((B, 1), jnp.int32)


def _sc_gather(scal_hbm, emb_hbm, cls_out, idx_v, rows_v, sem):
    cid = lax.axis_index("c")
    sid = lax.axis_index("s")
    wid = sid * 2 + cid

    @pl.when(wid < B)
    def _():
        b = wid
        pltpu.sync_copy(scal_hbm.at[b, pl.ds(0, CPAD)], idx_v)
        off = b * S
        idx_v[pl.ds(0, LANES)] = idx_v[pl.ds(0, LANES)] + off
        idx_v[pl.ds(LANES, LANES)] = idx_v[pl.ds(LANES, LANES)] + off
        pltpu.async_copy(emb_hbm.at[idx_v], rows_v, sem).wait()
        pltpu.sync_copy(rows_v.at[pl.ds(0, CROWS)], cls_out.at[b])


def _make_sc_call():
    return pl.kernel(
        _sc_gather,
        out_type=jax.ShapeDtypeStruct((B, CROWS, H), jnp.float32),
        mesh=plsc.VectorSubcoreMesh(core_axis_name="c", subcore_axis_name="s"),
        scratch_types=[
            pltpu.VMEM((CPAD,), jnp.int32),
            pltpu.VMEM((CPAD, H), jnp.float32),
            pltpu.SemaphoreType.DMA,
        ],
    )


def _tc_body(scal_ref, emb_ref, attn_ref, wt_ref, wc_ref, u_ref, acc, cnt):
    b = pl.program_id(0)
    k = pl.program_id(1)
    ts = scal_ref[b, 32]
    eos = scal_ref[b, 33]

    @pl.when(k == 0)
    def _():
        acc[...] = jnp.zeros_like(acc)
        cnt[0] = 0

    @pl.when(k * CS <= eos)
    def _():
        posr = k * CS + lax.broadcasted_iota(jnp.int32, (1, CS), 1)
        att = attn_ref[0, 0, pl.ds(k * CS, CS)]
        m = ((posr >= ts) & (posr < eos) & (posr < ts + MAX_TEXT)
             & (att[None, :] != 0))
        mf = m.astype(jnp.float32)
        chunk = emb_ref[...].reshape(CS, H)
        acc[...] += jnp.dot(mf, chunk, preferred_element_type=jnp.float32)
        cnt[0] += jnp.sum(m.astype(jnp.int32))

    @pl.when(k == NCHUNK - 1)
    def _():
        denom = cnt[0].astype(jnp.float32) + 1e-8
        pooled = acc[...] / denom
        text_rep = jnp.dot(pooled, wt_ref[...],
                           preferred_element_type=jnp.float32)
        u_ref[pl.ds(b, 1), :] = lax.dot_general(
            text_rep, wc_ref[...], (((1,), (1,)), ((), ())),
            preferred_element_type=jnp.float32)


def _logits_body(u_ref, cls_ref, scal_ref, scale_ref, out_ref):
    scale = scale_ref[0, 0]
    for b in range(B):
        clsb = cls_ref[b]
        lo = lax.dot_general(u_ref[b:b + 1, :], clsb,
                             (((1,), (1,)), ((), ())),
                             preferred_element_type=jnp.float32)
        cio = lax.broadcasted_iota(jnp.int32, (1, CROWS), 1)
        lo = jnp.where(cio < scal_ref[b, 34], lo, 0.0) * scale
        pad = jnp.zeros((1, 128 - CROWS), jnp.float32)
        out_ref[b:b + 1, :] = jnp.concatenate([lo, pad], axis=1)


def kernel(token_embeds, input_ids, attention_mask, W_text, W_class,
           logit_scale):
    ids = input_ids.astype(jnp.int32)
    attn = attention_mask.astype(jnp.int32)
    emb_flat = token_embeds.reshape(B * S, H)

    scal = pl.pallas_call(
        _idx_body,
        out_shape=jax.ShapeDtypeStruct((B, SCW), jnp.int32),
    )(ids, attn)

    cls_rows = _make_sc_call()(scal, emb_flat)

    attn3 = attn.reshape(B, 1, S)
    scale2d = logit_scale.astype(jnp.float32).reshape(1, 1)

    grid_spec = pltpu.PrefetchScalarGridSpec(
        num_scalar_prefetch=1,
        grid=(B, NCHUNK),
        in_specs=[
            pl.BlockSpec((1, CS, H),
                         lambda b, k, sc: (b, jnp.minimum(k, sc[b, 33] // CS), 0)),
            pl.BlockSpec((1, 1, S), lambda b, k, sc: (b, 0, 0)),
            pl.BlockSpec((H, H), lambda b, k, sc: (0, 0)),
            pl.BlockSpec((H, H), lambda b, k, sc: (0, 0)),
        ],
        out_specs=pl.BlockSpec((8, H), lambda b, k, sc: (0, 0)),
        scratch_shapes=[
            pltpu.VMEM((1, H), jnp.float32),
            pltpu.SMEM((1,), jnp.int32),
        ],
    )
    u = pl.pallas_call(
        _tc_body,
        grid_spec=grid_spec,
        out_shape=jax.ShapeDtypeStruct((8, H), jnp.float32),
        compiler_params=pltpu.CompilerParams(
            dimension_semantics=("arbitrary", "arbitrary")),
    )(scal, token_embeds, attn3, W_text, W_class)

    out = pl.pallas_call(
        _logits_body,
        in_specs=[
            pl.BlockSpec((8, H), lambda: (0, 0)),
            pl.BlockSpec((B, CROWS, H), lambda: (0, 0, 0)),
            pl.BlockSpec(memory_space=pltpu.SMEM),
            pl.BlockSpec(memory_space=pltpu.SMEM),
        ],
        out_shape=jax.ShapeDtypeStruct((8, 128), jnp.float32),
    )(u, cls_rows, scal, scale2d)
    return out[:B, :C]


# speedup vs baseline: 1.3102x; 1.3102x over previous
import jax
import jax.numpy as jnp
from jax import lax
from jax.experimental import pallas as pl
from jax.experimental.pallas import tpu as pltpu
from jax.experimental.pallas import tpu_sc as plsc

B, S, H = 8, 4096, 1024
CLASS_ID, TEXT_ID = 1, 2
C = 16 + B - 1
CROWS = 24
CPAD = 32
SCW = 40
LANES = 16
MAX_TEXT = S - (16 * 8 + 2)
CS = 512
NCHUNK = S // CS


def _idx_body(ids_ref, attn_ref, scal_ref):
    ids = ids_ref[...]
    attn = attn_ref[...]
    pos = lax.broadcasted_iota(jnp.int32, (B, S), 1)
    cmask = ids == CLASS_ID
    ncl = jnp.sum(jnp.where(cmask, 1, 0), axis=1, keepdims=True)
    ts = jnp.min(jnp.where(ids == TEXT_ID, pos, S), axis=1, keepdims=True)
    ts = jnp.where(ts >= S, 0, ts)
    eos = jnp.max(jnp.where(attn != 0, pos, -1), axis=1, keepdims=True)
    eos = jnp.where(eos < 0, S - 1, eos)
    prev = jnp.full((B, 1), -1, jnp.int32)
    for c in range(CROWS):
        cur = jnp.min(jnp.where(cmask & (pos > prev), pos, S), axis=1,
                      keepdims=True)
        scal_ref[:, c:c + 1] = jnp.where(cur < S, cur, 0)
        prev = cur
    for c in range(CROWS, 32):
        scal_ref[:, c:c + 1] = jnp.zeros((B, 1), jnp.int32)
    scal_ref[:, 32:33] = ts
    scal_ref[:, 33:34] = eos
    scal_ref[:, 34:35] = ncl
    for c in range(35, SCW):
        scal_ref[:, c:c + 1] = jnp.zeros((B, 1), jnp.int32)


def _sc_gather(scal_hbm, emb_hbm, cls_out, idx_v, rows_v, sem):
    cid = lax.axis_index("c")
    sid = lax.axis_index("s")
    wid = sid * 2 + cid

    @pl.when(wid < B)
    def _():
        b = wid
        pltpu.sync_copy(scal_hbm.at[b, pl.ds(0, CPAD)], idx_v)
        off = b * S
        idx_v[pl.ds(0, LANES)] = idx_v[pl.ds(0, LANES)] + off
        idx_v[pl.ds(LANES, LANES)] = idx_v[pl.ds(LANES, LANES)] + off
        pltpu.async_copy(emb_hbm.at[idx_v], rows_v, sem).wait()
        pltpu.sync_copy(rows_v.at[pl.ds(0, CROWS)], cls_out.at[b])


def _make_sc_call():
    return pl.kernel(
        _sc_gather,
        out_type=jax.ShapeDtypeStruct((B, CROWS, H), jnp.float32),
        mesh=plsc.VectorSubcoreMesh(core_axis_name="c", subcore_axis_name="s"),
        scratch_types=[
            pltpu.VMEM((CPAD,), jnp.int32),
            pltpu.VMEM((CPAD, H), jnp.float32),
            pltpu.SemaphoreType.DMA,
        ],
    )


def _tc_body(scal_ref, emb_ref, attn_ref, wt_ref, wc_ref, u_ref, acc, cnt):
    b = pl.program_id(0)
    k = pl.program_id(1)
    ts = scal_ref[b, 32]
    eos = scal_ref[b, 33]

    @pl.when(k == 0)
    def _():
        acc[...] = jnp.zeros_like(acc)
        cnt[0] = 0

    @pl.when(k * CS <= eos)
    def _():
        posr = k * CS + lax.broadcasted_iota(jnp.int32, (1, CS), 1)
        att = attn_ref[0, 0, pl.ds(k * CS, CS)]
        m = ((posr >= ts) & (posr < eos) & (posr < ts + MAX_TEXT)
             & (att[None, :] != 0))
        mf = m.astype(jnp.float32)
        chunk = emb_ref[...].reshape(CS, H)
        acc[...] += jnp.dot(mf, chunk, preferred_element_type=jnp.float32)
        cnt[0] += jnp.sum(m.astype(jnp.int32))

    @pl.when(k == NCHUNK - 1)
    def _():
        denom = cnt[0].astype(jnp.float32) + 1e-8
        pooled = acc[...] / denom
        text_rep = jnp.dot(pooled, wt_ref[...],
                           preferred_element_type=jnp.float32)
        u_ref[pl.ds(b, 1), :] = lax.dot_general(
            text_rep, wc_ref[...], (((1,), (1,)), ((), ())),
            preferred_element_type=jnp.float32)


def _logits_body(u_ref, cls_ref, scal_ref, scale_ref, out_ref):
    scale = scale_ref[0, 0]
    for b in range(B):
        clsb = cls_ref[b]
        lo = lax.dot_general(u_ref[b:b + 1, :], clsb,
                             (((1,), (1,)), ((), ())),
                             preferred_element_type=jnp.float32)
        cio = lax.broadcasted_iota(jnp.int32, (1, CROWS), 1)
        lo = jnp.where(cio < scal_ref[b, 34], lo, 0.0) * scale
        pad = jnp.zeros((1, 128 - CROWS), jnp.float32)
        out_ref[b:b + 1, :] = jnp.concatenate([lo, pad], axis=1)


def kernel(token_embeds, input_ids, attention_mask, W_text, W_class,
           logit_scale):
    ids = input_ids.astype(jnp.int32)
    attn = attention_mask.astype(jnp.int32)
    emb_flat = token_embeds.reshape(B * S, H)

    scal = pl.pallas_call(
        _idx_body,
        out_shape=jax.ShapeDtypeStruct((B, SCW), jnp.int32),
    )(ids, attn)

    cls_rows = _make_sc_call()(scal, emb_flat)

    attn3 = attn.reshape(B, 1, S)
    scale2d = logit_scale.astype(jnp.float32).reshape(1, 1)

    grid_spec = pltpu.PrefetchScalarGridSpec(
        num_scalar_prefetch=1,
        grid=(B, NCHUNK),
        in_specs=[
            pl.BlockSpec((1, CS, H),
                         lambda b, k, sc: (b, jnp.minimum(k, sc[b, 33] // CS), 0)),
            pl.BlockSpec((1, 1, S), lambda b, k, sc: (b, 0, 0)),
            pl.BlockSpec((H, H), lambda b, k, sc: (0, 0)),
            pl.BlockSpec((H, H), lambda b, k, sc: (0, 0)),
        ],
        out_specs=pl.BlockSpec((8, H), lambda b, k, sc: (0, 0)),
        scratch_shapes=[
            pltpu.VMEM((1, H), jnp.float32),
            pltpu.SMEM((1,), jnp.int32),
        ],
    )
    u = pl.pallas_call(
        _tc_body,
        grid_spec=grid_spec,
        out_shape=jax.ShapeDtypeStruct((8, H), jnp.float32),
        compiler_params=pltpu.CompilerParams(
            dimension_semantics=("arbitrary", "arbitrary")),
    )(scal, token_embeds, attn3, W_text, W_class)

    out = pl.pallas_call(
        _logits_body,
        in_specs=[
            pl.BlockSpec((8, H), lambda: (0, 0)),
            pl.BlockSpec((B, CROWS, H), lambda: (0, 0, 0)),
            pl.BlockSpec(memory_space=pltpu.SMEM),
            pl.BlockSpec(memory_space=pltpu.SMEM),
        ],
        out_shape=jax.ShapeDtypeStruct((8, 128), jnp.float32),
    )(u, cls_rows, scal, scale2d)
    return out[:B, :C]


# speedup vs baseline: 1.4889x; 1.1364x over previous
import jax
import jax.numpy as jnp
from jax import lax
from jax.experimental import pallas as pl
from jax.experimental.pallas import tpu as pltpu
from jax.experimental.pallas import tpu_sc as plsc

B, S, H = 8, 4096, 1024
CLASS_ID, TEXT_ID = 1, 2
C = 16 + B - 1
CROWS = 24
CPAD = 32
SCW = 40
LANES = 16
MAX_TEXT = S - (16 * 8 + 2)
CS = 1024
NCHUNK = S // CS


def _idx_body(ids_ref, attn_ref, scal_ref):
    ids = ids_ref[...]
    attn = attn_ref[...]
    pos = lax.broadcasted_iota(jnp.int32, (B, S), 1)
    cmask = ids == CLASS_ID
    ncl = jnp.sum(jnp.where(cmask, 1, 0), axis=1, keepdims=True)
    ts = jnp.min(jnp.where(ids == TEXT_ID, pos, S), axis=1, keepdims=True)
    ts = jnp.where(ts >= S, 0, ts)
    eos = jnp.max(jnp.where(attn != 0, pos, -1), axis=1, keepdims=True)
    eos = jnp.where(eos < 0, S - 1, eos)
    prev = jnp.full((B, 1), -1, jnp.int32)
    for c in range(CROWS):
        cur = jnp.min(jnp.where(cmask & (pos > prev), pos, S), axis=1,
                      keepdims=True)
        scal_ref[:, c:c + 1] = jnp.where(cur < S, cur, 0)
        prev = cur
    for c in range(CROWS, 32):
        scal_ref[:, c:c + 1] = jnp.zeros((B, 1), jnp.int32)
    scal_ref[:, 32:33] = ts
    scal_ref[:, 33:34] = eos
    scal_ref[:, 34:35] = ncl
    for c in range(35, SCW):
        scal_ref[:, c:c + 1] = jnp.zeros((B, 1), jnp.int32)


def _sc_gather(scal_hbm, emb_hbm, cls_out, idx_v, rows_v, sem):
    cid = lax.axis_index("c")
    sid = lax.axis_index("s")
    wid = sid * 2 + cid

    @pl.when(wid < B)
    def _():
        b = wid
        pltpu.sync_copy(scal_hbm.at[b, pl.ds(0, CPAD)], idx_v)
        off = b * S
        idx_v[pl.ds(0, LANES)] = idx_v[pl.ds(0, LANES)] + off
        idx_v[pl.ds(LANES, LANES)] = idx_v[pl.ds(LANES, LANES)] + off
        pltpu.async_copy(emb_hbm.at[idx_v], rows_v, sem).wait()
        pltpu.sync_copy(rows_v.at[pl.ds(0, CROWS)], cls_out.at[b])


def _make_sc_call():
    return pl.kernel(
        _sc_gather,
        out_type=jax.ShapeDtypeStruct((B, CROWS, H), jnp.float32),
        mesh=plsc.VectorSubcoreMesh(core_axis_name="c", subcore_axis_name="s"),
        scratch_types=[
            pltpu.VMEM((CPAD,), jnp.int32),
            pltpu.VMEM((CPAD, H), jnp.float32),
            pltpu.SemaphoreType.DMA,
        ],
    )


def _tc_body(scal_ref, emb_ref, attn_ref, wt_ref, wc_ref, u_ref, acc, cnt):
    b = pl.program_id(0)
    k = pl.program_id(1)
    ts = scal_ref[b, 32]
    eos = scal_ref[b, 33]

    @pl.when(k == 0)
    def _():
        acc[...] = jnp.zeros_like(acc)
        cnt[0] = 0

    @pl.when(k * CS <= eos)
    def _():
        posr = k * CS + lax.broadcasted_iota(jnp.int32, (1, CS), 1)
        att = attn_ref[0, 0, pl.ds(k * CS, CS)]
        m = ((posr >= ts) & (posr < eos) & (posr < ts + MAX_TEXT)
             & (att[None, :] != 0))
        mf = m.astype(jnp.float32)
        chunk = emb_ref[...].reshape(CS, H)
        acc[...] += jnp.dot(mf, chunk, preferred_element_type=jnp.float32)
        cnt[0] += jnp.sum(m.astype(jnp.int32))

    @pl.when(k == NCHUNK - 1)
    def _():
        denom = cnt[0].astype(jnp.float32) + 1e-8
        pooled = acc[...] / denom
        text_rep = jnp.dot(pooled, wt_ref[...],
                           preferred_element_type=jnp.float32)
        u_ref[pl.ds(b, 1), :] = lax.dot_general(
            text_rep, wc_ref[...], (((1,), (1,)), ((), ())),
            preferred_element_type=jnp.float32)


def _logits_body(u_ref, cls_ref, scal_ref, scale_ref, out_ref):
    scale = scale_ref[0, 0]
    for b in range(B):
        clsb = cls_ref[b]
        lo = lax.dot_general(u_ref[b:b + 1, :], clsb,
                             (((1,), (1,)), ((), ())),
                             preferred_element_type=jnp.float32)
        cio = lax.broadcasted_iota(jnp.int32, (1, CROWS), 1)
        lo = jnp.where(cio < scal_ref[b, 34], lo, 0.0) * scale
        pad = jnp.zeros((1, 128 - CROWS), jnp.float32)
        out_ref[b:b + 1, :] = jnp.concatenate([lo, pad], axis=1)


def kernel(token_embeds, input_ids, attention_mask, W_text, W_class,
           logit_scale):
    ids = input_ids.astype(jnp.int32)
    attn = attention_mask.astype(jnp.int32)
    emb_flat = token_embeds.reshape(B * S, H)

    scal = pl.pallas_call(
        _idx_body,
        out_shape=jax.ShapeDtypeStruct((B, SCW), jnp.int32),
    )(ids, attn)

    cls_rows = _make_sc_call()(scal, emb_flat)

    attn3 = attn.reshape(B, 1, S)
    scale2d = logit_scale.astype(jnp.float32).reshape(1, 1)

    grid_spec = pltpu.PrefetchScalarGridSpec(
        num_scalar_prefetch=1,
        grid=(B, NCHUNK),
        in_specs=[
            pl.BlockSpec((1, CS, H),
                         lambda b, k, sc: (b, jnp.minimum(k, sc[b, 33] // CS), 0)),
            pl.BlockSpec((1, 1, S), lambda b, k, sc: (b, 0, 0)),
            pl.BlockSpec((H, H), lambda b, k, sc: (0, 0)),
            pl.BlockSpec((H, H), lambda b, k, sc: (0, 0)),
        ],
        out_specs=pl.BlockSpec((8, H), lambda b, k, sc: (0, 0)),
        scratch_shapes=[
            pltpu.VMEM((1, H), jnp.float32),
            pltpu.SMEM((1,), jnp.int32),
        ],
    )
    u = pl.pallas_call(
        _tc_body,
        grid_spec=grid_spec,
        out_shape=jax.ShapeDtypeStruct((8, H), jnp.float32),
        compiler_params=pltpu.CompilerParams(
            dimension_semantics=("arbitrary", "arbitrary")),
    )(scal, token_embeds, attn3, W_text, W_class)

    out = pl.pallas_call(
        _logits_body,
        in_specs=[
            pl.BlockSpec((8, H), lambda: (0, 0)),
            pl.BlockSpec((B, CROWS, H), lambda: (0, 0, 0)),
            pl.BlockSpec(memory_space=pltpu.SMEM),
            pl.BlockSpec(memory_space=pltpu.SMEM),
        ],
        out_shape=jax.ShapeDtypeStruct((8, 128), jnp.float32),
    )(u, cls_rows, scal, scale2d)
    return out[:B, :C]
